# Initial kernel scaffold; baseline (speedup 1.0000x reference)
import jax
import jax.numpy as jnp
from jax.experimental import pallas as pl


def kernel(in_ids, size_tensor):
    raise NotImplementedError("write your pallas kernel here")


# speedup vs baseline: 17.3343x; 17.3343x over previous
import jax
import jax.numpy as jnp
from jax import lax
from jax.experimental import pallas as pl
from jax.experimental.pallas import tpu as pltpu
from jax.experimental.pallas import tpu_sc as plsc

_MASK = 1_000_000
_HALF = _MASK // 2
_NIDS = 100_000
_NSUB = 16
_NCORE = 2
_PAD_TO = 102_400
_PER_W = _PAD_TO // _NSUB
_CH = 128
_N_CH = _PER_W // _CH
_SLICE = 31_248
_TAIL = _HALF - _NSUB * _SLICE
_ZB = _SLICE // 3


def _scatter_body(ids_hbm, out_hbm, half, idx_v, sidx, sval, zbuf, stage_v, sem):
    c = lax.axis_index("c")
    s = lax.axis_index("s")
    base = c * _HALF

    idx_dma = pltpu.async_copy(ids_hbm.at[pl.ds(s * _PER_W, _PER_W)], idx_v, sem)

    zvec = jnp.zeros((16,), jnp.int32)

    @pl.loop(0, _ZB // 16)
    def _(i):
        zbuf[pl.ds(i * 16, 16)] = zvec

    for k in range(_SLICE // _ZB):
        pltpu.sync_copy(zbuf, half.at[pl.ds(s * _SLICE + k * _ZB, _ZB)])

    @pl.when(s == _NSUB - 1)
    def _():
        pltpu.sync_copy(zbuf.at[pl.ds(0, _TAIL)],
                        half.at[pl.ds(_NSUB * _SLICE, _TAIL)])

    idx_dma.wait()
    plsc.subcore_barrier()

    @pl.loop(0, _N_CH)
    def _(jc):
        for k in range(_CH // 16):
            v = idx_v[pl.ds(jc * _CH + k * 16, 16)]
            local = v - base
            inr = (local >= 0) & (local < _HALF)
            safe = jnp.where(inr, local, v & 0x3FFFF)
            sidx[jc, pl.ds(k * 16, 16)] = safe
            sval[jc, pl.ds(k * 16, 16)] = inr.astype(jnp.int32)

    @pl.loop(0, _N_CH)
    def _(jc):
        pltpu.sync_copy(sval.at[jc], half.at[sidx.at[jc]], add=True)

    plsc.subcore_barrier()

    pltpu.sync_copy(half.at[pl.ds(s * _SLICE, _SLICE)], stage_v)
    pltpu.sync_copy(stage_v, out_hbm.at[pl.ds(base + s * _SLICE, _SLICE)])

    @pl.when(s == _NSUB - 1)
    def _():
        pltpu.sync_copy(half.at[pl.ds(_NSUB * _SLICE, _TAIL)],
                        stage_v.at[pl.ds(0, _TAIL)])
        pltpu.sync_copy(stage_v.at[pl.ds(0, _TAIL)],
                        out_hbm.at[pl.ds(base + _NSUB * _SLICE, _TAIL)])


def kernel(in_ids, size_tensor):
    assert size_tensor.shape[0] == _MASK and in_ids.shape[0] == _NIDS
    ids = in_ids.astype(jnp.int32)
    pad = -1 - jnp.arange(_PAD_TO - _NIDS, dtype=jnp.int32)
    ids = jnp.concatenate([ids, pad])

    mesh = plsc.VectorSubcoreMesh(core_axis_name="c", subcore_axis_name="s",
                                  num_cores=_NCORE, num_subcores=_NSUB)
    run = pl.kernel(
        _scatter_body,
        out_type=jax.ShapeDtypeStruct((_MASK,), jnp.int32),
        mesh=mesh,
        compiler_params=pltpu.CompilerParams(needs_layout_passes=False),
        scratch_types=[
            pltpu.VMEM_SHARED((_HALF,), jnp.int32),
            pltpu.VMEM((_PER_W,), jnp.int32),
            pltpu.VMEM((_N_CH, _CH), jnp.int32),
            pltpu.VMEM((_N_CH, _CH), jnp.int32),
            pltpu.VMEM((_ZB,), jnp.int32),
            pltpu.VMEM((_SLICE,), jnp.int32),
            pltpu.SemaphoreType.DMA,
        ],
    )
    return run(ids).astype(jnp.bool_)
